# Initial kernel scaffold; baseline (speedup 1.0000x reference)
import jax
import jax.numpy as jnp
from jax.experimental import pallas as pl


def kernel(sim_mat):
    raise NotImplementedError("write your pallas kernel here")


# speedup vs baseline: 300.6599x; 300.6599x over previous
import jax
import jax.numpy as jnp
from jax.experimental import pallas as pl

_B, _C, _N = 8, 190, 16384
_NB = 2048


def _conf_body(x_ref, out_ref):
    x = x_ref[0]
    m1 = jnp.max(x, axis=0)
    is_max = x == m1[None, :]
    cnt = jnp.sum(is_max.astype(jnp.float32), axis=0)
    neg = jnp.float32(-jnp.inf)
    m2c = jnp.max(jnp.where(is_max, neg, x), axis=0)
    m2 = jnp.where(cnt > 1.0, m1, m2c)
    conf = jnp.exp(1.0 - m1 / (m2 + 1e-8))
    out_ref[0, 0, :] = conf


def kernel(sim_mat):
    nblk = _N // _NB
    conf = pl.pallas_call(
        _conf_body,
        grid=(_B, nblk),
        in_specs=[pl.BlockSpec((1, _C, _NB), lambda b, n: (b, 0, n))],
        out_specs=pl.BlockSpec((1, 1, _NB), lambda b, n: (b * nblk + n, 0, 0)),
        out_shape=jax.ShapeDtypeStruct((_B * nblk, 1, _NB), jnp.float32),
    )(sim_mat)
    return jnp.mean(conf.reshape(_B, nblk * _NB), axis=-1)


# speedup vs baseline: 337.1745x; 1.1214x over previous
import jax
import jax.numpy as jnp
from jax.experimental import pallas as pl

_B, _C, _N = 8, 190, 16384
_NB = 4096


def _conf_body(x_ref, out_ref):
    x = x_ref[0]
    m1 = jnp.max(x, axis=0)
    is_max = x == m1[None, :]
    cnt = jnp.sum(is_max.astype(jnp.float32), axis=0)
    neg = jnp.float32(-jnp.inf)
    m2c = jnp.max(jnp.where(is_max, neg, x), axis=0)
    m2 = jnp.where(cnt > 1.0, m1, m2c)
    conf = jnp.exp(1.0 - m1 / (m2 + 1e-8))
    out_ref[0, 0, :] = conf


def kernel(sim_mat):
    nblk = _N // _NB
    conf = pl.pallas_call(
        _conf_body,
        grid=(_B, nblk),
        in_specs=[pl.BlockSpec((1, _C, _NB), lambda b, n: (b, 0, n))],
        out_specs=pl.BlockSpec((1, 1, _NB), lambda b, n: (b * nblk + n, 0, 0)),
        out_shape=jax.ShapeDtypeStruct((_B * nblk, 1, _NB), jnp.float32),
    )(sim_mat)
    return jnp.mean(conf.reshape(_B, nblk * _NB), axis=-1)


# speedup vs baseline: 357.9123x; 1.0615x over previous
import jax
import jax.numpy as jnp
from jax.experimental import pallas as pl

_B, _C, _N = 8, 190, 16384
_NB = 8192


def _conf_body(x_ref, out_ref):
    x = x_ref[0]
    m1 = jnp.max(x, axis=0)
    is_max = x == m1[None, :]
    cnt = jnp.sum(is_max.astype(jnp.float32), axis=0)
    neg = jnp.float32(-jnp.inf)
    m2c = jnp.max(jnp.where(is_max, neg, x), axis=0)
    m2 = jnp.where(cnt > 1.0, m1, m2c)
    conf = jnp.exp(1.0 - m1 / (m2 + 1e-8))
    out_ref[0, 0, :] = conf


def kernel(sim_mat):
    nblk = _N // _NB
    conf = pl.pallas_call(
        _conf_body,
        grid=(_B, nblk),
        in_specs=[pl.BlockSpec((1, _C, _NB), lambda b, n: (b, 0, n))],
        out_specs=pl.BlockSpec((1, 1, _NB), lambda b, n: (b * nblk + n, 0, 0)),
        out_shape=jax.ShapeDtypeStruct((_B * nblk, 1, _NB), jnp.float32),
    )(sim_mat)
    return jnp.mean(conf.reshape(_B, nblk * _NB), axis=-1)
